# baseline (speedup 1.0000x reference)
import jax
import jax.numpy as jnp
from jax import lax
from jax.experimental import pallas as pl
from jax.experimental.pallas import tpu as pltpu

_N = 59616
_B = 256
_V = 256
_W = 512
_NB = 117
_NPAD = _NB * _W
_RC = 8
_DT_POS = 2
_DT_NEG = 12
_BIG = 1 << 30

_KS1 = 42
_KS2 = 0 ^ _KS1 ^ 0x1BD11BDA
_ROTS = ((13, 15, 26, 6), (17, 29, 16, 24))


def _threefry_bits(x1):
    ks = (jnp.uint32(0), jnp.uint32(_KS1), jnp.uint32(_KS2))
    x0 = jnp.broadcast_to(ks[0], x1.shape)
    x1 = x1 + ks[1]
    for i in range(5):
        for r in _ROTS[i % 2]:
            x0 = x0 + x1
            x1 = (lax.shift_left(x1, jnp.uint32(r))
                  | lax.shift_right_logical(x1, jnp.uint32(32 - r)))
            x1 = x1 ^ x0
        x0 = x0 + ks[(i + 1) % 3]
        x1 = x1 + ks[(i + 2) % 3] + jnp.uint32(i + 1)
    return x0 ^ x1


def _body(node_ref, time_ref, adjp_ref, bnode_ref, btime_ref, bidx_ref,
          out_ref, neigh_scr):
    bnode = bnode_ref[:, :]

    iota_vb = lax.broadcasted_iota(jnp.int32, (_B, _V), 1)
    ohb = (bnode == iota_vb).astype(jnp.float32)
    adjb = jnp.dot(ohb, adjp_ref[:, :], preferred_element_type=jnp.float32)

    iota_vw = lax.broadcasted_iota(jnp.int32, (_V, _W), 0)
    e = (lax.broadcasted_iota(jnp.int32, (_B // 8, _B), 1)
         - 8 * lax.broadcasted_iota(jnp.int32, (_B // 8, _B), 0))
    pack8 = jnp.where((e >= 0) & (e < 8),
                      lax.shift_left(1, jnp.maximum(e, 0)), 0
                      ).astype(jnp.float32)

    def pa(blk, _):
        node_b = node_ref[blk, :, :]
        oh = (iota_vw == node_b).astype(jnp.float32)
        nf = jnp.dot(adjb, oh, preferred_element_type=jnp.float32)
        nfb = (nf > 0.5).astype(jnp.float32)
        by = jnp.dot(pack8, nfb, preferred_element_type=jnp.float32)
        neigh_scr[:, 0, pl.ds(blk * _W, _W)] = by.astype(jnp.int32)
        return ()

    lax.fori_loop(0, _NB, pa, ())

    iota_w_u = lax.broadcasted_iota(jnp.uint32, (1, _W), 1)
    iota_w_i = lax.broadcasted_iota(jnp.int32, (1, _W), 1)
    iota_r_u = lax.broadcasted_iota(jnp.uint32, (_RC, 1), 0)
    iota_r_i = lax.broadcasted_iota(jnp.int32, (_RC, 1), 0)

    def make_chunk(c):
        def chunk(rc, _):
            b0 = rc * _RC
            r0u = jnp.uint32(c * _B) + b0.astype(jnp.uint32)
            flatbase = (r0u + iota_r_u) * jnp.uint32(_N)
            bn = bnode_ref[pl.ds(b0, _RC), :]
            bt = btime_ref[pl.ds(b0, _RC), :]
            bi = bidx_ref[pl.ds(b0, _RC), :]

            def pb(blk, carry):
                ak, ai = carry
                node_b = node_ref[blk, :, :]
                time_b = time_ref[blk, :, :]
                nglob = blk * _W + iota_w_i

                dt = jnp.abs(time_b - bt)
                if c == 0:
                    same = node_b == bn
                    mask = same & (dt <= _DT_POS) & (nglob != bi)
                else:
                    byrow = neigh_scr[rc, :, pl.ds(blk * _W, _W)]
                    neigh = (lax.shift_right_logical(byrow, iota_r_i)
                             & 1) == 1
                    if c == 1:
                        mask = neigh & (dt == 0)
                    else:
                        same = node_b == bn
                        mask = ((same & (dt >= _DT_NEG) & (nglob != bi))
                                | (~neigh & (dt > _DT_POS) & (nglob < _N)))

                def with_key(ak, ai):
                    x1 = flatbase + (jnp.uint32(blk * _W) + iota_w_u)
                    key = lax.shift_right_logical(
                        _threefry_bits(x1), jnp.uint32(9)).astype(jnp.int32)
                    mk = jnp.where(mask, key, -1)
                    better = mk > ak
                    return (jnp.where(better, mk, ak),
                            jnp.where(better,
                                      jnp.broadcast_to(nglob, ai.shape), ai))

                if c == 2:
                    return with_key(ak, ai)
                return lax.cond(jnp.any(mask), with_key,
                                lambda ak, ai: (ak, ai), ak, ai)

            ak0 = jnp.full((_RC, _W), -1, jnp.int32)
            ai0 = jnp.full((_RC, _W), _BIG, jnp.int32)
            ak, ai = lax.fori_loop(0, _NB, pb, (ak0, ai0),
                                   unroll=4 if c == 2 else 2)

            m = jnp.max(ak, axis=1, keepdims=True)
            first = jnp.min(jnp.where(ak == m, ai, _BIG), axis=1,
                            keepdims=True)
            out_ref[pl.ds(c * _B + b0, _RC), :] = jnp.where(m >= 0, first, -1)
            return ()
        return chunk

    for c in range(3):
        lax.fori_loop(0, _B // _RC, make_chunk(c), ())


@jax.jit
def kernel(node_ids, time_ids, adj_matrix, batch_indices, num_pos, num_neg):
    n = node_ids.shape[0]
    b_node = jnp.take(node_ids, batch_indices, axis=0).astype(jnp.int32)
    b_time = jnp.take(time_ids, batch_indices, axis=0).astype(jnp.int32)

    node_p = jnp.pad(node_ids.astype(jnp.int32), (0, _NPAD - n),
                     constant_values=255).reshape(_NB, 1, _W)
    time_p = jnp.pad(time_ids.astype(jnp.int32), (0, _NPAD - n),
                     constant_values=0).reshape(_NB, 1, _W)
    adj_p = jnp.pad(adj_matrix.astype(jnp.float32),
                    ((0, _V - adj_matrix.shape[0]),
                     (0, _V - adj_matrix.shape[1])))

    best = pl.pallas_call(
        _body,
        out_shape=jax.ShapeDtypeStruct((3 * _B, 1), jnp.int32),
        in_specs=[
            pl.BlockSpec((_NB, 1, _W), lambda: (0, 0, 0)),
            pl.BlockSpec((_NB, 1, _W), lambda: (0, 0, 0)),
            pl.BlockSpec((_V, _V), lambda: (0, 0)),
            pl.BlockSpec((_B, 1), lambda: (0, 0)),
            pl.BlockSpec((_B, 1), lambda: (0, 0)),
            pl.BlockSpec((_B, 1), lambda: (0, 0)),
        ],
        out_specs=pl.BlockSpec((3 * _B, 1), lambda: (0, 0)),
        scratch_shapes=[
            pltpu.VMEM((_B // 8, 1, _NPAD), jnp.int32),
        ],
    )(node_p, time_p, adj_p,
      b_node.reshape(_B, 1), b_time.reshape(_B, 1),
      batch_indices.astype(jnp.int32).reshape(_B, 1))

    best = best.reshape(3, _B)
    p1 = jnp.where(num_pos >= 1, best[0], -1)
    p2 = jnp.where(num_pos >= 1, best[1], -1)
    nn = jnp.where(num_neg >= 1, best[2], -1)

    src = batch_indices.astype(jnp.int32)
    pos_j = jnp.stack([p1, p2], axis=1).astype(jnp.int32)
    pos_pairs = jnp.stack(
        [jnp.broadcast_to(src[:, None], pos_j.shape), pos_j], axis=-1)
    pos_pairs = jnp.where(pos_j[..., None] >= 0, pos_pairs, -1)

    neg_j = nn[:, None].astype(jnp.int32)
    neg_pairs = jnp.stack(
        [jnp.broadcast_to(src[:, None], neg_j.shape), neg_j], axis=-1)
    neg_pairs = jnp.where(neg_j[..., None] >= 0, neg_pairs, -1)
    return pos_pairs, neg_pairs


# speedup vs baseline: 2.0018x; 2.0018x over previous
import jax
import jax.numpy as jnp
from jax import lax
from jax.experimental import pallas as pl
from jax.experimental.pallas import tpu as pltpu

_N = 59616
_B = 256
_V = 256
_W = 512
_NB = 117
_NPAD = _NB * _W
_RC = 8
_DT_POS = 2
_DT_NEG = 12
_BIG = 1 << 30

_KS1 = 42
_KS2 = 0 ^ _KS1 ^ 0x1BD11BDA
_ROTS = ((13, 15, 26, 6), (17, 29, 16, 24))


def _threefry_bits(x1):
    ks = (jnp.uint32(0), jnp.uint32(_KS1), jnp.uint32(_KS2))
    x0 = jnp.broadcast_to(ks[0], x1.shape)
    x1 = x1 + ks[1]
    for i in range(5):
        for r in _ROTS[i % 2]:
            x0 = x0 + x1
            x1 = (lax.shift_left(x1, jnp.uint32(r))
                  | lax.shift_right_logical(x1, jnp.uint32(32 - r)))
            x1 = x1 ^ x0
        x0 = x0 + ks[(i + 1) % 3]
        x1 = x1 + ks[(i + 2) % 3] + jnp.uint32(i + 1)
    return x0 ^ x1


def _body(node_ref, time_ref, adjp_ref, bnode_ref, btime_ref, bidx_ref,
          out_ref, neigh_scr):
    bnode = bnode_ref[:, :]

    iota_vb = lax.broadcasted_iota(jnp.int32, (_B, _V), 1)
    ohb = (bnode == iota_vb).astype(jnp.float32)
    adjb = jnp.dot(ohb, adjp_ref[:, :], preferred_element_type=jnp.float32)

    iota_vw = lax.broadcasted_iota(jnp.int32, (_V, _W), 0)
    e = (lax.broadcasted_iota(jnp.int32, (_B // 8, _B), 1)
         - 8 * lax.broadcasted_iota(jnp.int32, (_B // 8, _B), 0))
    pack8 = jnp.where((e >= 0) & (e < 8),
                      lax.shift_left(1, jnp.maximum(e, 0)), 0
                      ).astype(jnp.float32)

    def pa(blk, _):
        node_b = node_ref[blk, :, :]
        oh = (iota_vw == node_b).astype(jnp.float32)
        nf = jnp.dot(adjb, oh, preferred_element_type=jnp.float32)
        nfb = (nf > 0.5).astype(jnp.float32)
        by = jnp.dot(pack8, nfb, preferred_element_type=jnp.float32)
        neigh_scr[:, 0, pl.ds(blk * _W, _W)] = by.astype(jnp.int32)
        return ()

    lax.fori_loop(0, _NB, pa, ())

    iota_w_u = lax.broadcasted_iota(jnp.uint32, (1, _W), 1)
    iota_w_i = lax.broadcasted_iota(jnp.int32, (1, _W), 1)
    iota_r_u = lax.broadcasted_iota(jnp.uint32, (_RC, 1), 0)
    iota_r_i = lax.broadcasted_iota(jnp.int32, (_RC, 1), 0)

    def make_chunk(c):
        def chunk(rc, _):
            b0 = rc * _RC
            r0u = jnp.uint32(c * _B) + b0.astype(jnp.uint32)
            flatbase = (r0u + iota_r_u) * jnp.uint32(_N)
            bn = bnode_ref[pl.ds(b0, _RC), :]
            bt = btime_ref[pl.ds(b0, _RC), :]
            bi = bidx_ref[pl.ds(b0, _RC), :]

            def pb(blk, carry):
                ak, ai = carry
                node_b = node_ref[blk, :, :]
                time_b = time_ref[blk, :, :]
                nglob = blk * _W + iota_w_i

                dt = jnp.abs(time_b - bt)
                if c == 0:
                    same = node_b == bn
                    mask = same & (dt <= _DT_POS) & (nglob != bi)
                else:
                    byrow = neigh_scr[rc, :, pl.ds(blk * _W, _W)]
                    neigh = (lax.shift_right_logical(byrow, iota_r_i)
                             & 1) == 1
                    if c == 1:
                        mask = neigh & (dt == 0)
                    else:
                        same = node_b == bn
                        mask = ((same & (dt >= _DT_NEG) & (nglob != bi))
                                | (~neigh & (dt > _DT_POS) & (nglob < _N)))

                def with_key(ak, ai):
                    x1 = flatbase + (jnp.uint32(blk * _W) + iota_w_u)
                    key = lax.shift_right_logical(
                        _threefry_bits(x1), jnp.uint32(9)).astype(jnp.int32)
                    mk = jnp.where(mask, key, -1)
                    better = mk > ak
                    return (jnp.where(better, mk, ak),
                            jnp.where(better,
                                      jnp.broadcast_to(nglob, ai.shape), ai))

                return with_key(ak, ai)

            ak0 = jnp.full((_RC, _W), -1, jnp.int32)
            ai0 = jnp.full((_RC, _W), _BIG, jnp.int32)
            ak, ai = lax.fori_loop(0, _NB, pb, (ak0, ai0),
                                   unroll=4 if c == 2 else 2)

            m = jnp.max(ak, axis=1, keepdims=True)
            first = jnp.min(jnp.where(ak == m, ai, _BIG), axis=1,
                            keepdims=True)
            out_ref[pl.ds(c * _B + b0, _RC), :] = jnp.where(m >= 0, first, -1)
            return ()
        return chunk

    for c in range(3):
        lax.fori_loop(0, _B // _RC, make_chunk(c), ())


@jax.jit
def kernel(node_ids, time_ids, adj_matrix, batch_indices, num_pos, num_neg):
    n = node_ids.shape[0]
    b_node = jnp.take(node_ids, batch_indices, axis=0).astype(jnp.int32)
    b_time = jnp.take(time_ids, batch_indices, axis=0).astype(jnp.int32)

    node_p = jnp.pad(node_ids.astype(jnp.int32), (0, _NPAD - n),
                     constant_values=255).reshape(_NB, 1, _W)
    time_p = jnp.pad(time_ids.astype(jnp.int32), (0, _NPAD - n),
                     constant_values=0).reshape(_NB, 1, _W)
    adj_p = jnp.pad(adj_matrix.astype(jnp.float32),
                    ((0, _V - adj_matrix.shape[0]),
                     (0, _V - adj_matrix.shape[1])))

    best = pl.pallas_call(
        _body,
        out_shape=jax.ShapeDtypeStruct((3 * _B, 1), jnp.int32),
        in_specs=[
            pl.BlockSpec((_NB, 1, _W), lambda: (0, 0, 0)),
            pl.BlockSpec((_NB, 1, _W), lambda: (0, 0, 0)),
            pl.BlockSpec((_V, _V), lambda: (0, 0)),
            pl.BlockSpec((_B, 1), lambda: (0, 0)),
            pl.BlockSpec((_B, 1), lambda: (0, 0)),
            pl.BlockSpec((_B, 1), lambda: (0, 0)),
        ],
        out_specs=pl.BlockSpec((3 * _B, 1), lambda: (0, 0)),
        scratch_shapes=[
            pltpu.VMEM((_B // 8, 1, _NPAD), jnp.int32),
        ],
    )(node_p, time_p, adj_p,
      b_node.reshape(_B, 1), b_time.reshape(_B, 1),
      batch_indices.astype(jnp.int32).reshape(_B, 1))

    best = best.reshape(3, _B)
    p1 = jnp.where(num_pos >= 1, best[0], -1)
    p2 = jnp.where(num_pos >= 1, best[1], -1)
    nn = jnp.where(num_neg >= 1, best[2], -1)

    src = batch_indices.astype(jnp.int32)
    pos_j = jnp.stack([p1, p2], axis=1).astype(jnp.int32)
    pos_pairs = jnp.stack(
        [jnp.broadcast_to(src[:, None], pos_j.shape), pos_j], axis=-1)
    pos_pairs = jnp.where(pos_j[..., None] >= 0, pos_pairs, -1)

    neg_j = nn[:, None].astype(jnp.int32)
    neg_pairs = jnp.stack(
        [jnp.broadcast_to(src[:, None], neg_j.shape), neg_j], axis=-1)
    neg_pairs = jnp.where(neg_j[..., None] >= 0, neg_pairs, -1)
    return pos_pairs, neg_pairs


# speedup vs baseline: 2.0785x; 1.0383x over previous
import jax
import jax.numpy as jnp
from jax import lax
from jax.experimental import pallas as pl
from jax.experimental.pallas import tpu as pltpu

_N = 59616
_B = 256
_V = 256
_W = 512
_NB = 117
_NPAD = _NB * _W
_RC = 8
_DT_POS = 2
_DT_NEG = 12
_BIG = 1 << 30

_KS1 = 42
_KS2 = 0 ^ _KS1 ^ 0x1BD11BDA
_ROTS = ((13, 15, 26, 6), (17, 29, 16, 24))


def _threefry_bits(x1):
    ks = (jnp.uint32(0), jnp.uint32(_KS1), jnp.uint32(_KS2))
    x0 = jnp.broadcast_to(ks[0], x1.shape)
    x1 = x1 + ks[1]
    for i in range(5):
        for r in _ROTS[i % 2]:
            x0 = x0 + x1
            x1 = (lax.shift_left(x1, jnp.uint32(r))
                  | lax.shift_right_logical(x1, jnp.uint32(32 - r)))
            x1 = x1 ^ x0
        x0 = x0 + ks[(i + 1) % 3]
        x1 = x1 + ks[(i + 2) % 3] + jnp.uint32(i + 1)
    return x0 ^ x1


def _body(node_ref, time_ref, adjp_ref, bnode_ref, btime_ref, bidx_ref,
          out_ref, neigh_scr):
    bnode = bnode_ref[:, :]

    iota_vb = lax.broadcasted_iota(jnp.int32, (_B, _V), 1)
    ohb = (bnode == iota_vb).astype(jnp.float32)
    adjb = jnp.dot(ohb, adjp_ref[:, :], preferred_element_type=jnp.float32)

    iota_vw = lax.broadcasted_iota(jnp.int32, (_V, _W), 0)
    e = (lax.broadcasted_iota(jnp.int32, (_B // 8, _B), 1)
         - 8 * lax.broadcasted_iota(jnp.int32, (_B // 8, _B), 0))
    pack8 = jnp.where((e >= 0) & (e < 8),
                      lax.shift_left(1, jnp.maximum(e, 0)), 0
                      ).astype(jnp.float32)

    def pa(blk, _):
        node_b = node_ref[blk, :, :]
        oh = (iota_vw == node_b).astype(jnp.float32)
        nf = jnp.dot(adjb, oh, preferred_element_type=jnp.float32)
        nfb = (nf > 0.5).astype(jnp.float32)
        by = jnp.dot(pack8, nfb, preferred_element_type=jnp.float32)
        neigh_scr[:, 0, pl.ds(blk * _W, _W)] = by.astype(jnp.int32)
        return ()

    lax.fori_loop(0, _NB, pa, ())

    iota_w_u = lax.broadcasted_iota(jnp.uint32, (1, _W), 1)
    iota_w_i = lax.broadcasted_iota(jnp.int32, (1, _W), 1)
    iota_r_u = lax.broadcasted_iota(jnp.uint32, (_RC, 1), 0)
    iota_r_i = lax.broadcasted_iota(jnp.int32, (_RC, 1), 0)

    def make_chunk(c):
        def chunk(rc, _):
            b0 = rc * _RC
            r0u = jnp.uint32(c * _B) + b0.astype(jnp.uint32)
            flatbase = (r0u + iota_r_u) * jnp.uint32(_N)
            bn = bnode_ref[pl.ds(b0, _RC), :]
            bt = btime_ref[pl.ds(b0, _RC), :]
            bi = bidx_ref[pl.ds(b0, _RC), :]

            def pb(blk, carry):
                ak, ai = carry
                node_b = node_ref[blk, :, :]
                time_b = time_ref[blk, :, :]
                nglob = blk * _W + iota_w_i

                dt = jnp.abs(time_b - bt)
                if c == 0:
                    same = node_b == bn
                    mask = same & (dt <= _DT_POS) & (nglob != bi)
                else:
                    byrow = neigh_scr[rc, :, pl.ds(blk * _W, _W)]
                    neigh = (lax.shift_right_logical(byrow, iota_r_i)
                             & 1) == 1
                    if c == 1:
                        mask = neigh & (dt == 0)
                    else:
                        same = node_b == bn
                        mask = ((same & (dt >= _DT_NEG) & (nglob != bi))
                                | (~neigh & (dt > _DT_POS) & (nglob < _N)))

                def with_key(ak, ai):
                    x1 = flatbase + (jnp.uint32(blk * _W) + iota_w_u)
                    key = lax.shift_right_logical(
                        _threefry_bits(x1), jnp.uint32(9)).astype(jnp.int32)
                    mk = jnp.where(mask, key, -1)
                    better = mk > ak
                    return (jnp.where(better, mk, ak),
                            jnp.where(better,
                                      jnp.broadcast_to(nglob, ai.shape), ai))

                return with_key(ak, ai)

            ak0 = jnp.full((_RC, _W), -1, jnp.int32)
            ai0 = jnp.full((_RC, _W), _BIG, jnp.int32)
            ak, ai = lax.fori_loop(0, _NB, pb, (ak0, ai0), unroll=4)

            m = jnp.max(ak, axis=1, keepdims=True)
            first = jnp.min(jnp.where(ak == m, ai, _BIG), axis=1,
                            keepdims=True)
            out_ref[pl.ds(c * _B + b0, _RC), :] = jnp.where(m >= 0, first, -1)
            return ()
        return chunk

    for c in range(3):
        lax.fori_loop(0, _B // _RC, make_chunk(c), ())


@jax.jit
def kernel(node_ids, time_ids, adj_matrix, batch_indices, num_pos, num_neg):
    n = node_ids.shape[0]
    b_node = jnp.take(node_ids, batch_indices, axis=0).astype(jnp.int32)
    b_time = jnp.take(time_ids, batch_indices, axis=0).astype(jnp.int32)

    node_p = jnp.pad(node_ids.astype(jnp.int32), (0, _NPAD - n),
                     constant_values=255).reshape(_NB, 1, _W)
    time_p = jnp.pad(time_ids.astype(jnp.int32), (0, _NPAD - n),
                     constant_values=0).reshape(_NB, 1, _W)
    adj_p = jnp.pad(adj_matrix.astype(jnp.float32),
                    ((0, _V - adj_matrix.shape[0]),
                     (0, _V - adj_matrix.shape[1])))

    best = pl.pallas_call(
        _body,
        out_shape=jax.ShapeDtypeStruct((3 * _B, 1), jnp.int32),
        in_specs=[
            pl.BlockSpec((_NB, 1, _W), lambda: (0, 0, 0)),
            pl.BlockSpec((_NB, 1, _W), lambda: (0, 0, 0)),
            pl.BlockSpec((_V, _V), lambda: (0, 0)),
            pl.BlockSpec((_B, 1), lambda: (0, 0)),
            pl.BlockSpec((_B, 1), lambda: (0, 0)),
            pl.BlockSpec((_B, 1), lambda: (0, 0)),
        ],
        out_specs=pl.BlockSpec((3 * _B, 1), lambda: (0, 0)),
        scratch_shapes=[
            pltpu.VMEM((_B // 8, 1, _NPAD), jnp.int32),
        ],
    )(node_p, time_p, adj_p,
      b_node.reshape(_B, 1), b_time.reshape(_B, 1),
      batch_indices.astype(jnp.int32).reshape(_B, 1))

    best = best.reshape(3, _B)
    p1 = jnp.where(num_pos >= 1, best[0], -1)
    p2 = jnp.where(num_pos >= 1, best[1], -1)
    nn = jnp.where(num_neg >= 1, best[2], -1)

    src = batch_indices.astype(jnp.int32)
    pos_j = jnp.stack([p1, p2], axis=1).astype(jnp.int32)
    pos_pairs = jnp.stack(
        [jnp.broadcast_to(src[:, None], pos_j.shape), pos_j], axis=-1)
    pos_pairs = jnp.where(pos_j[..., None] >= 0, pos_pairs, -1)

    neg_j = nn[:, None].astype(jnp.int32)
    neg_pairs = jnp.stack(
        [jnp.broadcast_to(src[:, None], neg_j.shape), neg_j], axis=-1)
    neg_pairs = jnp.where(neg_j[..., None] >= 0, neg_pairs, -1)
    return pos_pairs, neg_pairs


# speedup vs baseline: 2.1677x; 1.0429x over previous
import jax
import jax.numpy as jnp
from jax import lax
from jax.experimental import pallas as pl
from jax.experimental.pallas import tpu as pltpu

_N = 59616
_B = 256
_V = 256
_W = 512
_NB = 117
_NPAD = _NB * _W
_RC = 8
_DT_POS = 2
_DT_NEG = 12
_BIG = 1 << 30

_KS1 = 42
_KS2 = 0 ^ _KS1 ^ 0x1BD11BDA
_ROTS = ((13, 15, 26, 6), (17, 29, 16, 24))


def _threefry_bits(x1):
    ks = (jnp.uint32(0), jnp.uint32(_KS1), jnp.uint32(_KS2))
    x0 = jnp.broadcast_to(ks[0], x1.shape)
    x1 = x1 + ks[1]
    for i in range(5):
        for r in _ROTS[i % 2]:
            x0 = x0 + x1
            x1 = (lax.shift_left(x1, jnp.uint32(r))
                  | lax.shift_right_logical(x1, jnp.uint32(32 - r)))
            x1 = x1 ^ x0
        x0 = x0 + ks[(i + 1) % 3]
        x1 = x1 + ks[(i + 2) % 3] + jnp.uint32(i + 1)
    return x0 ^ x1


def _body(node_ref, time_ref, adjp_ref, bnode_ref, btime_ref, bidx_ref,
          out_ref, neigh_scr):
    bnode = bnode_ref[:, :]

    iota_vb = lax.broadcasted_iota(jnp.int32, (_B, _V), 1)
    ohb = (bnode == iota_vb).astype(jnp.float32)
    adjb = jnp.dot(ohb, adjp_ref[:, :], preferred_element_type=jnp.float32)

    iota_vw = lax.broadcasted_iota(jnp.int32, (_V, _W), 0)
    e = (lax.broadcasted_iota(jnp.int32, (_B // 8, _B), 1)
         - 8 * lax.broadcasted_iota(jnp.int32, (_B // 8, _B), 0))
    pack8 = jnp.where((e >= 0) & (e < 8),
                      lax.shift_left(1, jnp.maximum(e, 0)), 0
                      ).astype(jnp.float32)

    def pa(blk, _):
        node_b = node_ref[blk, :, :]
        oh = (iota_vw == node_b).astype(jnp.float32)
        nf = jnp.dot(adjb, oh, preferred_element_type=jnp.float32)
        nfb = (nf > 0.5).astype(jnp.float32)
        by = jnp.dot(pack8, nfb, preferred_element_type=jnp.float32)
        neigh_scr[:, 0, pl.ds(blk * _W, _W)] = by.astype(jnp.int32)
        return ()

    lax.fori_loop(0, _NB, pa, ())

    iota_w_u = lax.broadcasted_iota(jnp.uint32, (1, _W), 1)
    iota_w_i = lax.broadcasted_iota(jnp.int32, (1, _W), 1)
    iota_r_u = lax.broadcasted_iota(jnp.uint32, (_RC, 1), 0)
    iota_r_i = lax.broadcasted_iota(jnp.int32, (_RC, 1), 0)

    def make_chunk(c):
        def chunk(rc, _):
            b0 = rc * _RC
            r0u = jnp.uint32(c * _B) + b0.astype(jnp.uint32)
            flatbase = (r0u + iota_r_u) * jnp.uint32(_N)
            bn = bnode_ref[pl.ds(b0, _RC), :]
            bt = btime_ref[pl.ds(b0, _RC), :]
            bi = bidx_ref[pl.ds(b0, _RC), :]

            def pb(blk, carry):
                ak, ai = carry
                node_b = node_ref[blk, :, :]
                time_b = time_ref[blk, :, :]
                nglob = blk * _W + iota_w_i

                dt = jnp.abs(time_b - bt)
                if c == 0:
                    same = node_b == bn
                    mask = same & (dt <= _DT_POS) & (nglob != bi)
                else:
                    byrow = neigh_scr[rc, :, pl.ds(blk * _W, _W)]
                    neigh = (lax.shift_right_logical(byrow, iota_r_i)
                             & 1) == 1
                    if c == 1:
                        mask = neigh & (dt == 0)
                    else:
                        same = node_b == bn
                        mask = ((same & (dt >= _DT_NEG) & (nglob != bi))
                                | (~neigh & (dt > _DT_POS) & (nglob < _N)))

                def with_key(ak, ai):
                    x1 = flatbase + (jnp.uint32(blk * _W) + iota_w_u)
                    key = lax.shift_right_logical(
                        _threefry_bits(x1), jnp.uint32(9)).astype(jnp.int32)
                    mk = jnp.where(mask, key, -1)
                    better = mk > ak
                    return (jnp.where(better, mk, ak),
                            jnp.where(better,
                                      jnp.broadcast_to(nglob, ai.shape), ai))

                return with_key(ak, ai)

            ak0 = jnp.full((_RC, _W), -1, jnp.int32)
            ai0 = jnp.full((_RC, _W), _BIG, jnp.int32)
            ak, ai = lax.fori_loop(0, _NB, pb, (ak0, ai0), unroll=9)

            m = jnp.max(ak, axis=1, keepdims=True)
            first = jnp.min(jnp.where(ak == m, ai, _BIG), axis=1,
                            keepdims=True)
            out_ref[pl.ds(c * _B + b0, _RC), :] = jnp.where(m >= 0, first, -1)
            return ()
        return chunk

    for c in range(3):
        lax.fori_loop(0, _B // _RC, make_chunk(c), ())


@jax.jit
def kernel(node_ids, time_ids, adj_matrix, batch_indices, num_pos, num_neg):
    n = node_ids.shape[0]
    b_node = jnp.take(node_ids, batch_indices, axis=0).astype(jnp.int32)
    b_time = jnp.take(time_ids, batch_indices, axis=0).astype(jnp.int32)

    node_p = jnp.pad(node_ids.astype(jnp.int32), (0, _NPAD - n),
                     constant_values=255).reshape(_NB, 1, _W)
    time_p = jnp.pad(time_ids.astype(jnp.int32), (0, _NPAD - n),
                     constant_values=0).reshape(_NB, 1, _W)
    adj_p = jnp.pad(adj_matrix.astype(jnp.float32),
                    ((0, _V - adj_matrix.shape[0]),
                     (0, _V - adj_matrix.shape[1])))

    best = pl.pallas_call(
        _body,
        out_shape=jax.ShapeDtypeStruct((3 * _B, 1), jnp.int32),
        in_specs=[
            pl.BlockSpec((_NB, 1, _W), lambda: (0, 0, 0)),
            pl.BlockSpec((_NB, 1, _W), lambda: (0, 0, 0)),
            pl.BlockSpec((_V, _V), lambda: (0, 0)),
            pl.BlockSpec((_B, 1), lambda: (0, 0)),
            pl.BlockSpec((_B, 1), lambda: (0, 0)),
            pl.BlockSpec((_B, 1), lambda: (0, 0)),
        ],
        out_specs=pl.BlockSpec((3 * _B, 1), lambda: (0, 0)),
        scratch_shapes=[
            pltpu.VMEM((_B // 8, 1, _NPAD), jnp.int32),
        ],
    )(node_p, time_p, adj_p,
      b_node.reshape(_B, 1), b_time.reshape(_B, 1),
      batch_indices.astype(jnp.int32).reshape(_B, 1))

    best = best.reshape(3, _B)
    p1 = jnp.where(num_pos >= 1, best[0], -1)
    p2 = jnp.where(num_pos >= 1, best[1], -1)
    nn = jnp.where(num_neg >= 1, best[2], -1)

    src = batch_indices.astype(jnp.int32)
    pos_j = jnp.stack([p1, p2], axis=1).astype(jnp.int32)
    pos_pairs = jnp.stack(
        [jnp.broadcast_to(src[:, None], pos_j.shape), pos_j], axis=-1)
    pos_pairs = jnp.where(pos_j[..., None] >= 0, pos_pairs, -1)

    neg_j = nn[:, None].astype(jnp.int32)
    neg_pairs = jnp.stack(
        [jnp.broadcast_to(src[:, None], neg_j.shape), neg_j], axis=-1)
    neg_pairs = jnp.where(neg_j[..., None] >= 0, neg_pairs, -1)
    return pos_pairs, neg_pairs
